# Initial kernel scaffold; baseline (speedup 1.0000x reference)
import functools

import jax
import jax.numpy as jnp
from jax import lax
from jax.experimental import pallas as pl
from jax.experimental.pallas import tpu as pltpu
from jax.experimental.pallas import tpu_sc as plsc

N = 10000
E = 320000
D = 128
NC, NS = 2, 16
NW = NC * NS
EW = E // NW
CH = 80
NCH = EW // CH
RPT = N // NS
ZR = 125
DEGW = 16

BM = 1000

_mesh = plsc.VectorSubcoreMesh(core_axis_name="c", subcore_axis_name="s")


@functools.partial(
    pl.kernel,
    out_type=jax.ShapeDtypeStruct((NC, N, DEGW), jnp.float32),
    mesh=_mesh,
    scratch_types=[
        pltpu.VMEM((NCH, CH), jnp.int32),
        pltpu.VMEM((CH, DEGW), jnp.float32),
        pltpu.VMEM((RPT, DEGW), jnp.float32),
        pltpu.VMEM_SHARED((N, DEGW), jnp.float32),
    ],
)
def _deg_kernel(dst_hbm, deg_hbm, idx_v, ones_v, zero_v, acc_sh):
    c = lax.axis_index("c")
    s = lax.axis_index("s")
    wid = s * NC + c

    @pl.loop(0, CH)
    def _(i):
        ones_v[i, :] = jnp.ones((DEGW,), jnp.float32)

    @pl.loop(0, RPT)
    def _(i):
        zero_v[i, :] = jnp.zeros((DEGW,), jnp.float32)

    pltpu.sync_copy(zero_v, acc_sh.at[pl.ds(s * RPT, RPT)])
    plsc.subcore_barrier()

    pltpu.sync_copy(dst_hbm.at[wid], idx_v)

    @pl.loop(0, NCH)
    def _(j):
        pltpu.sync_copy(ones_v, acc_sh.at[idx_v.at[j]], add=True)

    plsc.subcore_barrier()
    pltpu.sync_copy(acc_sh.at[pl.ds(s * RPT, RPT)],
                    deg_hbm.at[c, pl.ds(s * RPT, RPT)])


@functools.partial(
    pl.kernel,
    out_type=jax.ShapeDtypeStruct((NC, N, D), jnp.float32),
    mesh=_mesh,
    scratch_types=[
        pltpu.VMEM((NCH, CH), jnp.int32),
        pltpu.VMEM((NCH, CH), jnp.int32),
        pltpu.VMEM((CH, D), jnp.float32),
        pltpu.VMEM((ZR, D), jnp.float32),
        pltpu.VMEM_SHARED((N, D), jnp.float32),
    ],
)
def _agg_kernel(xws_hbm, src_hbm, dst_hbm, out_hbm,
                src_v, dst_v, rows_v, zero_v, acc_sh):
    c = lax.axis_index("c")
    s = lax.axis_index("s")
    wid = s * NC + c

    @pl.loop(0, ZR)
    def _(i):
        for k in range(D // 16):
            zero_v[i, pl.ds(k * 16, 16)] = jnp.zeros((16,), jnp.float32)

    @pl.loop(0, RPT // ZR)
    def _(k):
        pltpu.sync_copy(zero_v, acc_sh.at[pl.ds(s * RPT + k * ZR, ZR)])

    plsc.subcore_barrier()

    pltpu.sync_copy(src_hbm.at[wid], src_v)
    pltpu.sync_copy(dst_hbm.at[wid], dst_v)

    @pl.loop(0, NCH)
    def _(j):
        pltpu.sync_copy(xws_hbm.at[src_v.at[j]], rows_v)
        pltpu.sync_copy(rows_v, acc_sh.at[dst_v.at[j]], add=True)

    plsc.subcore_barrier()
    pltpu.sync_copy(acc_sh.at[pl.ds(s * RPT, RPT)],
                    out_hbm.at[c, pl.ds(s * RPT, RPT)])


def _dot(a, b):
    return lax.dot_general(a, b, (((1,), (0,)), ((), ())),
                           preferred_element_type=jnp.float32,
                           precision=lax.Precision.HIGHEST)


def _mm_body(x_ref, w_ref, o_ref):
    o_ref[...] = _dot(x_ref[...], w_ref[...])


_mm = pl.pallas_call(
    _mm_body,
    grid=(N // BM,),
    in_specs=[
        pl.BlockSpec((BM, D), lambda i: (i, 0)),
        pl.BlockSpec((D, D), lambda i: (0, 0)),
    ],
    out_specs=pl.BlockSpec((BM, D), lambda i: (i, 0)),
    out_shape=jax.ShapeDtypeStruct((N, D), jnp.float32),
)


def _scale_body(deg_ref, xw_ref, dinv_ref, xws_ref):
    d = deg_ref[0, :, 0:1] + deg_ref[1, :, 0:1] + 1.0
    dinv = lax.rsqrt(d)
    dinv_ref[...] = dinv
    xws_ref[...] = xw_ref[...] * dinv


_scale = pl.pallas_call(
    _scale_body,
    grid=(N // BM,),
    in_specs=[
        pl.BlockSpec((2, BM, DEGW), lambda i: (0, i, 0)),
        pl.BlockSpec((BM, D), lambda i: (i, 0)),
    ],
    out_specs=[
        pl.BlockSpec((BM, 1), lambda i: (i, 0)),
        pl.BlockSpec((BM, D), lambda i: (i, 0)),
    ],
    out_shape=[
        jax.ShapeDtypeStruct((N, 1), jnp.float32),
        jax.ShapeDtypeStruct((N, D), jnp.float32),
    ],
)


def _mid_body(dinv_ref, p_ref, xws1_ref, b1_ref, w2_ref, o_ref):
    dinv = dinv_ref[...]
    agg = p_ref[0] + p_ref[1] + xws1_ref[...]
    h = jnp.maximum(agg * dinv + b1_ref[...], 0.0)
    o_ref[...] = _dot(h, w2_ref[...]) * dinv


_mid = pl.pallas_call(
    _mid_body,
    grid=(N // BM,),
    in_specs=[
        pl.BlockSpec((BM, 1), lambda i: (i, 0)),
        pl.BlockSpec((2, BM, D), lambda i: (0, i, 0)),
        pl.BlockSpec((BM, D), lambda i: (i, 0)),
        pl.BlockSpec((1, D), lambda i: (0, 0)),
        pl.BlockSpec((D, D), lambda i: (0, 0)),
    ],
    out_specs=pl.BlockSpec((BM, D), lambda i: (i, 0)),
    out_shape=jax.ShapeDtypeStruct((N, D), jnp.float32),
)


def _out_body(dinv_ref, q_ref, xws2_ref, b2_ref, o_ref):
    agg = q_ref[0] + q_ref[1] + xws2_ref[...]
    o_ref[...] = agg * dinv_ref[...] + b2_ref[...]


_out = pl.pallas_call(
    _out_body,
    grid=(N // BM,),
    in_specs=[
        pl.BlockSpec((BM, 1), lambda i: (i, 0)),
        pl.BlockSpec((2, BM, D), lambda i: (0, i, 0)),
        pl.BlockSpec((BM, D), lambda i: (i, 0)),
        pl.BlockSpec((1, D), lambda i: (0, 0)),
    ],
    out_specs=pl.BlockSpec((BM, D), lambda i: (i, 0)),
    out_shape=jax.ShapeDtypeStruct((N, D), jnp.float32),
)


def kernel(x, edge_index, W1, b1, W2, b2):
    src = edge_index[0].astype(jnp.int32).reshape(NW, NCH, CH)
    dst = edge_index[1].astype(jnp.int32).reshape(NW, NCH, CH)
    b1r = b1.reshape(1, D)
    b2r = b2.reshape(1, D)

    deg_p = _deg_kernel(dst)
    xw1 = _mm(x, W1)
    dinv, xws1 = _scale(deg_p, xw1)
    p = _agg_kernel(xws1, src, dst)
    xws2 = _mid(dinv, p, xws1, b1r, W2)
    q = _agg_kernel(xws2, src, dst)
    return _out(dinv, q, xws2, b2r)


# speedup vs baseline: 5.6918x; 5.6918x over previous
import functools

import jax
import jax.numpy as jnp
from jax import lax
from jax.experimental import pallas as pl
from jax.experimental.pallas import tpu as pltpu
from jax.experimental.pallas import tpu_sc as plsc

N = 10000
E = 320000
D = 128
NC, NS = 2, 16
EWA = E // NS
CH = 80
NCHA = EWA // CH
PH = 2
QR = 2560
NQ = 2500
RQT = QR // NS
TRB = NQ
DEGW = 16

BM = 1000

_mesh = plsc.VectorSubcoreMesh(core_axis_name="c", subcore_axis_name="s")


@functools.partial(
    pl.kernel,
    out_type=jax.ShapeDtypeStruct((NC, PH * QR, DEGW), jnp.float32),
    mesh=_mesh,
    scratch_types=[
        pltpu.VMEM((NCHA, CH), jnp.int32),
        pltpu.VMEM((NCHA, CH), jnp.int32),
        pltpu.VMEM((CH, DEGW), jnp.float32),
        pltpu.VMEM((RQT, DEGW), jnp.float32),
        pltpu.VMEM_SHARED((QR, DEGW), jnp.float32),
    ],
)
def _deg_kernel(dst_hbm, deg_hbm, idx_v, idxm_v, ones_v, zero_v, acc_sh):
    c = lax.axis_index("c")
    s = lax.axis_index("s")

    @pl.loop(0, CH)
    def _(i):
        ones_v[i, :] = jnp.ones((DEGW,), jnp.float32)

    @pl.loop(0, RQT)
    def _(i):
        zero_v[i, :] = jnp.zeros((DEGW,), jnp.float32)

    pltpu.sync_copy(dst_hbm.at[s], idx_v)

    for p in range(PH):
        base = (2 * c + p) * NQ

        pltpu.sync_copy(zero_v, acc_sh.at[pl.ds(s * RQT, RQT)])
        plsc.subcore_barrier()

        @pl.loop(0, NCHA)
        def _(j):
            for g in range(CH // 16):
                sl = pl.ds(g * 16, 16)
                v = idx_v[j, sl]
                t = v - base
                bad = (t < 0) | (t >= NQ)
                idxm_v[j, sl] = jnp.where(bad, TRB + (v & 31), t)

        @pl.loop(0, NCHA)
        def _(j):
            pltpu.sync_copy(ones_v, acc_sh.at[idxm_v.at[j]], add=True)

        plsc.subcore_barrier()
        pltpu.sync_copy(acc_sh.at[pl.ds(s * RQT, RQT)], zero_v)
        pltpu.sync_copy(zero_v, deg_hbm.at[c, pl.ds(p * QR + s * RQT, RQT)])

        @pl.loop(0, RQT)
        def _(i):
            zero_v[i, :] = jnp.zeros((DEGW,), jnp.float32)

        plsc.subcore_barrier()


@functools.partial(
    pl.kernel,
    out_type=jax.ShapeDtypeStruct((NC, QR, D), jnp.float32),
    mesh=_mesh,
    scratch_types=[
        pltpu.VMEM((NCHA, CH), jnp.int32),
        pltpu.VMEM((NCHA, CH), jnp.int32),
        pltpu.VMEM((CH, D), jnp.float32),
        pltpu.VMEM((RQT, D), jnp.float32),
        pltpu.VMEM_SHARED((QR, D), jnp.float32),
    ],
)
def _agg_kernel(xws_hbm, src_hbm, dst_hbm, out_hbm,
                src_v, dst_v, rows_v, zero_v, acc_sh):
    c = lax.axis_index("c")
    s = lax.axis_index("s")
    base = (2 * c) * NQ

    @pl.loop(0, RQT)
    def _(i):
        for k in range(D // 16):
            zero_v[i, pl.ds(k * 16, 16)] = jnp.zeros((16,), jnp.float32)

    pltpu.sync_copy(zero_v, acc_sh.at[pl.ds(s * RQT, RQT)])

    pltpu.sync_copy(src_hbm.at[s], src_v)
    pltpu.sync_copy(dst_hbm.at[s], dst_v)

    @pl.loop(0, NCHA)
    def _(j):
        for g in range(CH // 16):
            sl = pl.ds(g * 16, 16)
            v = dst_v[j, sl]
            t = v - base
            bad = (t < 0) | (t >= NQ)
            dst_v[j, sl] = jnp.where(bad, TRB + (v & 31), t)

    plsc.subcore_barrier()

    @pl.loop(0, NCHA)
    def _(j):
        pltpu.sync_copy(xws_hbm.at[src_v.at[j]], rows_v)
        pltpu.sync_copy(rows_v, acc_sh.at[dst_v.at[j]], add=True)

    plsc.subcore_barrier()
    pltpu.sync_copy(acc_sh.at[pl.ds(s * RQT, RQT)],
                    out_hbm.at[c, pl.ds(s * RQT, RQT)])


def _stitch(a):
    return jnp.concatenate(
        [a[0, :NQ], a[0, QR:QR + NQ], a[1, :NQ], a[1, QR:QR + NQ]], axis=0)


def _dot(a, b):
    return lax.dot_general(a, b, (((1,), (0,)), ((), ())),
                           preferred_element_type=jnp.float32,
                           precision=lax.Precision.HIGHEST)


def _mm_body(x_ref, w_ref, o_ref):
    o_ref[...] = _dot(x_ref[...], w_ref[...])


_mm = pl.pallas_call(
    _mm_body,
    grid=(N // BM,),
    in_specs=[
        pl.BlockSpec((BM, D), lambda i: (i, 0)),
        pl.BlockSpec((D, D), lambda i: (0, 0)),
    ],
    out_specs=pl.BlockSpec((BM, D), lambda i: (i, 0)),
    out_shape=jax.ShapeDtypeStruct((N, D), jnp.float32),
)


def _scale_body(deg_ref, xw_ref, dinv_ref, xws_ref):
    d = deg_ref[:, 0:1] + 1.0
    dinv = lax.rsqrt(d)
    dinv_ref[...] = dinv
    xws_ref[...] = xw_ref[...] * dinv


_scale = pl.pallas_call(
    _scale_body,
    grid=(N // BM,),
    in_specs=[
        pl.BlockSpec((BM, DEGW), lambda i: (i, 0)),
        pl.BlockSpec((BM, D), lambda i: (i, 0)),
    ],
    out_specs=[
        pl.BlockSpec((BM, 1), lambda i: (i, 0)),
        pl.BlockSpec((BM, D), lambda i: (i, 0)),
    ],
    out_shape=[
        jax.ShapeDtypeStruct((N, 1), jnp.float32),
        jax.ShapeDtypeStruct((N, D), jnp.float32),
    ],
)


def _stage_body(dinv_ref, p_ref, xws_ref, b1_ref, b2_ref, w2_ref,
                nxt_ref, res_ref):
    dinv = dinv_ref[...]
    agg = p_ref[...] + xws_ref[...]
    pre = agg * dinv
    h = jnp.maximum(pre + b1_ref[...], 0.0)
    nxt_ref[...] = _dot(h, w2_ref[...]) * dinv
    res_ref[...] = pre + b2_ref[...]


_stage = pl.pallas_call(
    _stage_body,
    grid=(N // BM,),
    in_specs=[
        pl.BlockSpec((BM, 1), lambda i: (i, 0)),
        pl.BlockSpec((BM, D), lambda i: (i, 0)),
        pl.BlockSpec((BM, D), lambda i: (i, 0)),
        pl.BlockSpec((1, D), lambda i: (0, 0)),
        pl.BlockSpec((1, D), lambda i: (0, 0)),
        pl.BlockSpec((D, D), lambda i: (0, 0)),
    ],
    out_specs=[
        pl.BlockSpec((BM, D), lambda i: (i, 0)),
        pl.BlockSpec((BM, D), lambda i: (i, 0)),
    ],
    out_shape=[
        jax.ShapeDtypeStruct((N, D), jnp.float32),
        jax.ShapeDtypeStruct((N, D), jnp.float32),
    ],
)


def kernel(x, edge_index, W1, b1, W2, b2):
    src_a = edge_index[0].astype(jnp.int32).reshape(NS, NCHA, CH)
    dst_a = edge_index[1].astype(jnp.int32).reshape(NS, NCHA, CH)
    dst_sh = dst_a - jnp.int32(NQ)
    b1r = b1.reshape(1, D)
    b2r = b2.reshape(1, D)

    deg_q = _deg_kernel(dst_a)
    xw1 = _mm(x, W1)
    dinv, xws1 = _scale(_stitch(deg_q), xw1)

    def body(k, carry):
        xws, pfull, res = carry
        ph = k & 1
        dst_k = jnp.where(ph == 1, dst_sh, dst_a)
        pq = _agg_kernel(xws, src_a, dst_k)
        pfull = lax.dynamic_update_slice(pfull, pq[0, :NQ], (ph * NQ, 0))
        pfull = lax.dynamic_update_slice(pfull, pq[1, :NQ],
                                         ((2 + ph) * NQ, 0))

        def do_stage():
            nxt, out = _stage(dinv, pfull, xws, b1r, b2r, W2)
            return nxt, out

        xws, res = lax.cond(ph == 1, do_stage, lambda: (xws, res))
        return xws, pfull, res

    four = jnp.int32(4) + jnp.asarray(x[0, 0] * 0.0).astype(jnp.int32)
    _, _, res = lax.fori_loop(0, four, body, (xws1, xws1, xws1),
                              unroll=False)
    return res
